# baseline (speedup 1.0000x reference)
import functools

import jax
import jax.numpy as jnp
from jax import lax
from jax.experimental import pallas as pl
from jax.experimental.pallas import tpu as pltpu
from jax.experimental.pallas import tpu_sc as plsc

E = 8
TOP_K = 2
D = 1024
T = 1024
H = 1024
B = 2048

GATE_BB = 256

MB = 256
NBLK = 24
P_MAX = NBLK * MB
NSLOT = B * TOP_K


def _gating_body(x_ref, gw_ref, vals_ref, idx_ref):
    gw = gw_ref[0, :].astype(jnp.bfloat16).astype(jnp.float32)
    logits = []
    for e in range(E):
        xe = x_ref[e].astype(jnp.bfloat16).astype(jnp.float32)
        logits.append(jnp.sum(xe * gw[None, :], axis=1, keepdims=True))
    lg = jnp.concatenate(logits, axis=1)
    m = jnp.max(lg, axis=1, keepdims=True)
    ex = jnp.exp(lg - m)
    p = ex / jnp.sum(ex, axis=1, keepdims=True)

    lane = jax.lax.broadcasted_iota(jnp.int32, p.shape, 1)
    m1 = jnp.max(p, axis=1, keepdims=True)
    i1 = jnp.min(jnp.where(p == m1, lane, E), axis=1, keepdims=True)
    p2 = jnp.where(lane == i1, -jnp.inf, p)
    m2 = jnp.max(p2, axis=1, keepdims=True)
    i2 = jnp.min(jnp.where(p2 == m2, lane, E), axis=1, keepdims=True)

    vals_ref[...] = jnp.concatenate([m1, m2], axis=1)
    idx_ref[...] = jnp.concatenate([i1, i2], axis=1)


def _gating(x, gW):
    return pl.pallas_call(
        _gating_body,
        grid=(B // GATE_BB,),
        in_specs=[
            pl.BlockSpec((E, GATE_BB, D), lambda i: (0, i, 0)),
            pl.BlockSpec((1, D), lambda i: (0, 0)),
        ],
        out_specs=[
            pl.BlockSpec((GATE_BB, TOP_K), lambda i: (i, 0)),
            pl.BlockSpec((GATE_BB, TOP_K), lambda i: (i, 0)),
        ],
        out_shape=[
            jax.ShapeDtypeStruct((B, TOP_K), jnp.float32),
            jax.ShapeDtypeStruct((B, TOP_K), jnp.int32),
        ],
    )(x, gW.reshape(1, D))


_SPB = NSLOT // 16


def _route_gather_body(idx_hbm, xflat_hbm, xs_hbm, pos_hbm, meta_hbm, cnts_hbm,
                       idx_v, cntrow_v, counts_all_v,
                       pos2d_v, midx2d_v, meta_v, rows_v, sem):
    cid = lax.axis_index("c")
    sid = lax.axis_index("s")
    iota = lax.broadcasted_iota(jnp.int32, (16,), 0)
    zeros = jnp.zeros((16,), jnp.int32)
    base_slot = sid * _SPB

    pltpu.sync_copy(idx_hbm.at[pl.ds(base_slot, _SPB)], idx_v)

    cnt = zeros
    for ch in range(_SPB // 16):
        v = idx_v[pl.ds(ch * 16, 16)]
        for e in range(E):
            c = jnp.sum(jnp.where(v == e, 1, 0))
            cnt = cnt + jnp.where(iota == e, c, 0)
    cntrow_v[...] = cnt
    pltpu.sync_copy(cntrow_v, cnts_hbm.at[sid])
    plsc.subcore_barrier()

    pltpu.sync_copy(cnts_hbm, counts_all_v)
    tot = zeros
    pre = zeros
    for w2 in range(16):
        row = counts_all_v[w2]
        tot = tot + row
        pre = pre + jnp.where(w2 < sid, row, zeros)
    bc = jnp.where(iota < E, (tot + (MB - 1)) // MB, 0)
    cs = plsc.cumsum(bc)
    excl = cs - bc
    nact = jnp.sum(bc)
    base = excl * MB + pre

    for ch in range(2):
        ivec = iota + ch * 16
        ieff = jnp.minimum(ivec, nact - 1)
        be = zeros
        for e in range(E):
            cse = jnp.sum(jnp.where(iota == e, cs, 0))
            be = be + jnp.where(ieff >= cse, 1, 0)
        if ch == 1:
            be = jnp.where(iota == 15, nact, be)
        meta_v[pl.ds(ch * 16, 16)] = be

    @pl.when(jnp.logical_and(cid == 0, sid == 0))
    def _write_meta():
        pltpu.sync_copy(meta_v, meta_hbm)

    runcnt = zeros
    for ch in range(_SPB // 16):
        v = idx_v[pl.ds(ch * 16, 16)]
        token = (base_slot + ch * 16 + iota) >> 1
        p = zeros
        for e in range(E):
            mask = v == e
            mi = jnp.where(mask, 1, 0)
            prefix = plsc.cumsum(mi) - mi
            be_s = jnp.sum(jnp.where(iota == e, base + runcnt, 0))
            p = jnp.where(mask, be_s + prefix, p)
            runcnt = runcnt + jnp.where(iota == e, jnp.sum(mi), 0)
        pos2d_v[ch, :] = p
        midx2d_v[ch, :] = v * B + token

    @pl.when(cid == 0)
    def _write_pos():
        pltpu.sync_copy(pos2d_v, pos_hbm.at[pl.ds(sid * 16, 16)])

    for ch in range(8):
        j = cid * 8 + ch
        pltpu.async_copy(xflat_hbm.at[midx2d_v.at[j]], rows_v, sem).wait()
        pltpu.sync_copy(rows_v, xs_hbm.at[pos2d_v.at[j]])


def _route_gather(idx_flat, x_flat):
    mesh = plsc.VectorSubcoreMesh(core_axis_name="c", subcore_axis_name="s")
    f = pl.kernel(
        _route_gather_body,
        out_type=[
            jax.ShapeDtypeStruct((P_MAX, D), jnp.float32),
            jax.ShapeDtypeStruct((NSLOT // 16, 16), jnp.int32),
            jax.ShapeDtypeStruct((32,), jnp.int32),
            jax.ShapeDtypeStruct((16, 16), jnp.int32),
        ],
        mesh=mesh,
        compiler_params=pltpu.CompilerParams(needs_layout_passes=False),
        scratch_types=[
            pltpu.VMEM((_SPB,), jnp.int32),
            pltpu.VMEM((16,), jnp.int32),
            pltpu.VMEM((16, 16), jnp.int32),
            pltpu.VMEM((16, 16), jnp.int32),
            pltpu.VMEM((16, 16), jnp.int32),
            pltpu.VMEM((32,), jnp.int32),
            pltpu.VMEM((16, D), jnp.float32),
            pltpu.SemaphoreType.DMA,
        ],
    )
    return f(idx_flat, x_flat)


def _ffn_body(meta_ref, xs_ref, w1_ref, b1_ref, w2_ref, b2_ref, o_ref):
    i = pl.program_id(0)
    nact = meta_ref[31]

    @pl.when(i < nact)
    def _compute():
        h = jnp.maximum(
            jnp.dot(xs_ref[...], w1_ref[0], preferred_element_type=jnp.float32)
            + b1_ref[0],
            0.0,
        )
        o_ref[...] = (
            jnp.dot(h, w2_ref[0], preferred_element_type=jnp.float32) + b2_ref[0]
        )


def _ffn(meta, xs, W1, b1r, W2, b2r):
    grid_spec = pltpu.PrefetchScalarGridSpec(
        num_scalar_prefetch=1,
        grid=(NBLK,),
        in_specs=[
            pl.BlockSpec((MB, D), lambda i, m: (i, 0)),
            pl.BlockSpec((1, D, H), lambda i, m: (m[i], 0, 0)),
            pl.BlockSpec((1, 1, H), lambda i, m: (m[i], 0, 0)),
            pl.BlockSpec((1, H, T), lambda i, m: (m[i], 0, 0)),
            pl.BlockSpec((1, 1, T), lambda i, m: (m[i], 0, 0)),
        ],
        out_specs=pl.BlockSpec((MB, T), lambda i, m: (i, 0)),
    )
    return pl.pallas_call(
        _ffn_body,
        grid_spec=grid_spec,
        out_shape=jax.ShapeDtypeStruct((P_MAX, T), jnp.float32),
        compiler_params=pltpu.CompilerParams(
            dimension_semantics=("arbitrary",),
        ),
    )(meta, xs, W1, b1r, W2, b2r)


def _combine_body(o_hbm, pos_hbm, vals_hbm, out_hbm,
                  pos_v, val_v, ridx_v, orow_v, out_v, sem):
    cid = lax.axis_index("c")
    sid = lax.axis_index("s")
    wid = cid * 16 + sid
    iota = lax.broadcasted_iota(jnp.int32, (16,), 0)

    pltpu.sync_copy(pos_hbm.at[pl.ds(wid * 8, 8)], pos_v)
    pltpu.sync_copy(vals_hbm.at[pl.ds(wid * 128, 128)], val_v)

    for g in range(4):
        lt = g * 16 + iota
        i0 = 2 * lt
        i1 = 2 * lt + 1
        r0 = plsc.load_gather(pos_v, [i0 >> 4, i0 & 15])
        r1 = plsc.load_gather(pos_v, [i1 >> 4, i1 & 15])
        ridx_v[pl.ds(0, 16)] = r0
        ridx_v[pl.ds(16, 16)] = r1
        pltpu.async_copy(o_hbm.at[ridx_v], orow_v, sem).wait()

        def _tok(tt, _):
            s0 = jnp.broadcast_to(2 * (g * 16 + tt), (16,))
            v0 = plsc.load_gather(val_v, [s0])
            v1 = plsc.load_gather(val_v, [s0 + 1])
            for c in range(T // 16):
                a = orow_v[tt, pl.ds(c * 16, 16)]
                b = orow_v[tt + 16, pl.ds(c * 16, 16)]
                out_v[tt, pl.ds(c * 16, 16)] = v0 * a + v1 * b
            return _

        lax.fori_loop(0, 16, _tok, 0)
        pltpu.sync_copy(out_v, out_hbm.at[pl.ds(wid * 64 + g * 16, 16)])


def _combine(o_sorted, pos, vals_flat):
    mesh = plsc.VectorSubcoreMesh(core_axis_name="c", subcore_axis_name="s")
    f = pl.kernel(
        _combine_body,
        out_type=jax.ShapeDtypeStruct((B, T), jnp.float32),
        mesh=mesh,
        compiler_params=pltpu.CompilerParams(needs_layout_passes=False),
        scratch_types=[
            pltpu.VMEM((8, 16), jnp.int32),
            pltpu.VMEM((128,), jnp.float32),
            pltpu.VMEM((32,), jnp.int32),
            pltpu.VMEM((32, T), jnp.float32),
            pltpu.VMEM((16, T), jnp.float32),
            pltpu.SemaphoreType.DMA,
        ],
    )
    return f(o_sorted, pos, vals_flat)


@jax.jit
def kernel(x, gW, gb, W1, b1, W2, b2):
    del gb
    vals, idx = _gating(x, gW)
    xs, pos, meta, _cnts = _route_gather(idx.reshape(NSLOT), x.reshape(E * B, D))
    o_sorted = _ffn(meta, xs, W1, b1.reshape(E, 1, H), W2, b2.reshape(E, 1, T))
    out = _combine(o_sorted, pos, vals.reshape(NSLOT))
    return (out, vals)


# speedup vs baseline: 1.0599x; 1.0599x over previous
import functools

import jax
import jax.numpy as jnp
from jax import lax
from jax.experimental import pallas as pl
from jax.experimental.pallas import tpu as pltpu
from jax.experimental.pallas import tpu_sc as plsc

E = 8
TOP_K = 2
D = 1024
T = 1024
H = 1024
B = 2048

GATE_BB = 256

MB = 256
NBLK = 24
P_MAX = NBLK * MB
NSLOT = B * TOP_K


def _gating_body(x_ref, gw_ref, vals_ref, idx_ref):
    gw = gw_ref[0, :].astype(jnp.bfloat16).astype(jnp.float32)
    logits = []
    for e in range(E):
        xe = x_ref[e].astype(jnp.bfloat16).astype(jnp.float32)
        logits.append(jnp.sum(xe * gw[None, :], axis=1, keepdims=True))
    lg = jnp.concatenate(logits, axis=1)
    m = jnp.max(lg, axis=1, keepdims=True)
    ex = jnp.exp(lg - m)
    p = ex / jnp.sum(ex, axis=1, keepdims=True)

    lane = jax.lax.broadcasted_iota(jnp.int32, p.shape, 1)
    m1 = jnp.max(p, axis=1, keepdims=True)
    i1 = jnp.min(jnp.where(p == m1, lane, E), axis=1, keepdims=True)
    p2 = jnp.where(lane == i1, -jnp.inf, p)
    m2 = jnp.max(p2, axis=1, keepdims=True)
    i2 = jnp.min(jnp.where(p2 == m2, lane, E), axis=1, keepdims=True)

    vals_ref[...] = jnp.concatenate([m1, m2], axis=1)
    idx_ref[...] = jnp.concatenate([i1, i2], axis=1)


def _gating(x, gW):
    return pl.pallas_call(
        _gating_body,
        grid=(B // GATE_BB,),
        in_specs=[
            pl.BlockSpec((E, GATE_BB, D), lambda i: (0, i, 0)),
            pl.BlockSpec((1, D), lambda i: (0, 0)),
        ],
        out_specs=[
            pl.BlockSpec((GATE_BB, TOP_K), lambda i: (i, 0)),
            pl.BlockSpec((GATE_BB, TOP_K), lambda i: (i, 0)),
        ],
        out_shape=[
            jax.ShapeDtypeStruct((B, TOP_K), jnp.float32),
            jax.ShapeDtypeStruct((B, TOP_K), jnp.int32),
        ],
    )(x, gW.reshape(1, D))


_SPB = NSLOT // 16


def _route_gather_body(idx_hbm, xflat_hbm, xs_hbm, pos_hbm, meta_hbm, cnts_hbm,
                       idx_v, cntrow_v, counts_all_v,
                       pos2d_v, midx2d_v, meta_v, rows_v, sem0, sem1):
    cid = lax.axis_index("c")
    sid = lax.axis_index("s")
    iota = lax.broadcasted_iota(jnp.int32, (16,), 0)
    zeros = jnp.zeros((16,), jnp.int32)
    base_slot = sid * _SPB

    pltpu.sync_copy(idx_hbm.at[pl.ds(base_slot, _SPB)], idx_v)

    cnt = zeros
    for ch in range(_SPB // 16):
        v = idx_v[pl.ds(ch * 16, 16)]
        for e in range(E):
            c = jnp.sum(jnp.where(v == e, 1, 0))
            cnt = cnt + jnp.where(iota == e, c, 0)
    cntrow_v[...] = cnt
    pltpu.sync_copy(cntrow_v, cnts_hbm.at[sid])
    plsc.subcore_barrier()

    pltpu.sync_copy(cnts_hbm, counts_all_v)
    tot = zeros
    pre = zeros
    for w2 in range(16):
        row = counts_all_v[w2]
        tot = tot + row
        pre = pre + jnp.where(w2 < sid, row, zeros)
    bc = jnp.where(iota < E, (tot + (MB - 1)) // MB, 0)
    cs = plsc.cumsum(bc)
    excl = cs - bc
    nact = jnp.sum(bc)
    base = excl * MB + pre

    for ch in range(2):
        ivec = iota + ch * 16
        ieff = jnp.minimum(ivec, nact - 1)
        be = zeros
        for e in range(E):
            cse = jnp.sum(jnp.where(iota == e, cs, 0))
            be = be + jnp.where(ieff >= cse, 1, 0)
        if ch == 1:
            be = jnp.where(iota == 15, nact, be)
        meta_v[pl.ds(ch * 16, 16)] = be

    @pl.when(jnp.logical_and(cid == 0, sid == 0))
    def _write_meta():
        pltpu.sync_copy(meta_v, meta_hbm)

    runcnt = zeros
    for ch in range(_SPB // 16):
        v = idx_v[pl.ds(ch * 16, 16)]
        token = (base_slot + ch * 16 + iota) >> 1
        p = zeros
        for e in range(E):
            mask = v == e
            mi = jnp.where(mask, 1, 0)
            prefix = plsc.cumsum(mi) - mi
            be_s = jnp.sum(jnp.where(iota == e, base + runcnt, 0))
            p = jnp.where(mask, be_s + prefix, p)
            runcnt = runcnt + jnp.where(iota == e, jnp.sum(mi), 0)
        pos2d_v[ch, :] = p
        midx2d_v[ch, :] = v * B + token

    @pl.when(cid == 0)
    def _write_pos():
        pltpu.sync_copy(pos2d_v, pos_hbm.at[pl.ds(sid * 16, 16)])

    g0 = pltpu.make_async_copy(xflat_hbm.at[midx2d_v.at[cid * 8]], rows_v.at[0], sem0)
    g0.start()
    for ch in range(8):
        j = cid * 8 + ch
        cur = rows_v.at[ch % 2]
        pltpu.make_async_copy(xflat_hbm.at[midx2d_v.at[j]], cur, sem0 if ch % 2 == 0 else sem1).wait()
        if ch + 1 < 8:
            nxt = rows_v.at[(ch + 1) % 2]
            pltpu.make_async_copy(
                xflat_hbm.at[midx2d_v.at[j + 1]], nxt,
                sem0 if (ch + 1) % 2 == 0 else sem1).start()
        pltpu.sync_copy(cur, xs_hbm.at[pos2d_v.at[j]])


def _route_gather(idx_flat, x_flat):
    mesh = plsc.VectorSubcoreMesh(core_axis_name="c", subcore_axis_name="s")
    f = pl.kernel(
        _route_gather_body,
        out_type=[
            jax.ShapeDtypeStruct((P_MAX, D), jnp.float32),
            jax.ShapeDtypeStruct((NSLOT // 16, 16), jnp.int32),
            jax.ShapeDtypeStruct((32,), jnp.int32),
            jax.ShapeDtypeStruct((16, 16), jnp.int32),
        ],
        mesh=mesh,
        compiler_params=pltpu.CompilerParams(needs_layout_passes=False),
        scratch_types=[
            pltpu.VMEM((_SPB,), jnp.int32),
            pltpu.VMEM((16,), jnp.int32),
            pltpu.VMEM((16, 16), jnp.int32),
            pltpu.VMEM((16, 16), jnp.int32),
            pltpu.VMEM((16, 16), jnp.int32),
            pltpu.VMEM((32,), jnp.int32),
            pltpu.VMEM((2, 16, D), jnp.float32),
            pltpu.SemaphoreType.DMA,
            pltpu.SemaphoreType.DMA,
        ],
    )
    return f(idx_flat, x_flat)


def _ffn_body(meta_ref, xs_ref, w1_ref, b1_ref, w2_ref, b2_ref, o_ref):
    i = pl.program_id(0)
    nact = meta_ref[31]

    @pl.when(i < nact)
    def _compute():
        h = jnp.maximum(
            jnp.dot(xs_ref[...], w1_ref[0], preferred_element_type=jnp.float32)
            + b1_ref[0],
            0.0,
        )
        o_ref[...] = (
            jnp.dot(h, w2_ref[0], preferred_element_type=jnp.float32) + b2_ref[0]
        )


def _ffn(meta, xs, W1, b1r, W2, b2r):
    grid_spec = pltpu.PrefetchScalarGridSpec(
        num_scalar_prefetch=1,
        grid=(NBLK,),
        in_specs=[
            pl.BlockSpec((MB, D), lambda i, m: (i, 0)),
            pl.BlockSpec((1, D, H), lambda i, m: (m[i], 0, 0)),
            pl.BlockSpec((1, 1, H), lambda i, m: (m[i], 0, 0)),
            pl.BlockSpec((1, H, T), lambda i, m: (m[i], 0, 0)),
            pl.BlockSpec((1, 1, T), lambda i, m: (m[i], 0, 0)),
        ],
        out_specs=pl.BlockSpec((MB, T), lambda i, m: (i, 0)),
    )
    return pl.pallas_call(
        _ffn_body,
        grid_spec=grid_spec,
        out_shape=jax.ShapeDtypeStruct((P_MAX, T), jnp.float32),
        compiler_params=pltpu.CompilerParams(
            dimension_semantics=("arbitrary",),
        ),
    )(meta, xs, W1, b1r, W2, b2r)


def _combine_body(o_hbm, pos_hbm, vals_hbm, out_hbm,
                  pos_v, val_v, ridx_v, orow_v, out_v, sem0, sem1):
    cid = lax.axis_index("c")
    sid = lax.axis_index("s")
    wid = cid * 16 + sid
    iota = lax.broadcasted_iota(jnp.int32, (16,), 0)

    pltpu.sync_copy(pos_hbm.at[pl.ds(wid * 8, 8)], pos_v)
    pltpu.sync_copy(vals_hbm.at[pl.ds(wid * 128, 128)], val_v)

    for g in range(4):
        lt = g * 16 + iota
        i0 = 2 * lt
        i1 = 2 * lt + 1
        ridx_v[g, pl.ds(0, 16)] = plsc.load_gather(pos_v, [i0 >> 4, i0 & 15])
        ridx_v[g, pl.ds(16, 16)] = plsc.load_gather(pos_v, [i1 >> 4, i1 & 15])

    pltpu.make_async_copy(o_hbm.at[ridx_v.at[0]], orow_v.at[0], sem0).start()
    for g in range(4):
        buf = g % 2
        pltpu.make_async_copy(
            o_hbm.at[ridx_v.at[g]], orow_v.at[buf],
            sem0 if buf == 0 else sem1).wait()
        if g + 1 < 4:
            pltpu.make_async_copy(
                o_hbm.at[ridx_v.at[g + 1]], orow_v.at[(g + 1) % 2],
                sem0 if (g + 1) % 2 == 0 else sem1).start()

        def _tok(tt, _):
            s0 = jnp.broadcast_to(2 * (g * 16 + tt), (16,))
            v0 = plsc.load_gather(val_v, [s0])
            v1 = plsc.load_gather(val_v, [s0 + 1])
            for c in range(T // 16):
                a = orow_v[buf, tt, pl.ds(c * 16, 16)]
                b = orow_v[buf, tt + 16, pl.ds(c * 16, 16)]
                out_v[tt, pl.ds(c * 16, 16)] = v0 * a + v1 * b
            return _

        lax.fori_loop(0, 16, _tok, 0)
        pltpu.sync_copy(out_v, out_hbm.at[pl.ds(wid * 64 + g * 16, 16)])


def _combine(o_sorted, pos, vals_flat):
    mesh = plsc.VectorSubcoreMesh(core_axis_name="c", subcore_axis_name="s")
    f = pl.kernel(
        _combine_body,
        out_type=jax.ShapeDtypeStruct((B, T), jnp.float32),
        mesh=mesh,
        compiler_params=pltpu.CompilerParams(needs_layout_passes=False),
        scratch_types=[
            pltpu.VMEM((8, 16), jnp.int32),
            pltpu.VMEM((128,), jnp.float32),
            pltpu.VMEM((4, 32), jnp.int32),
            pltpu.VMEM((2, 32, T), jnp.float32),
            pltpu.VMEM((16, T), jnp.float32),
            pltpu.SemaphoreType.DMA,
            pltpu.SemaphoreType.DMA,
        ],
    )
    return f(o_sorted, pos, vals_flat)


@jax.jit
def kernel(x, gW, gb, W1, b1, W2, b2):
    del gb
    vals, idx = _gating(x, gW)
    xs, pos, meta, _cnts = _route_gather(idx.reshape(NSLOT), x.reshape(E * B, D))
    o_sorted = _ffn(meta, xs, W1, b1.reshape(E, 1, H), W2, b2.reshape(E, 1, T))
    out = _combine(o_sorted, pos, vals.reshape(NSLOT))
    return (out, vals)
